# baseline (device time: 7161 ns/iter reference)
import jax
import jax.numpy as jnp
from jax import lax
from jax.experimental import pallas as pl
from jax.experimental.pallas import tpu as pltpu

N_DEV = 4
N_HALF = 2


def kernel(x):
    m_per, n = x.shape
    nh = n // N_HALF

    def body(x_ref, out_ref, comm_ref, send_sems, recv_sems):
        my_pos = lax.axis_index("i")

        barrier_sem = pltpu.get_barrier_semaphore()
        for d in range(1, N_DEV):
            pl.semaphore_signal(
                barrier_sem, inc=1,
                device_id=((my_pos + d) % N_DEV,),
                device_id_type=pl.DeviceIdType.MESH,
            )
        pl.semaphore_wait(barrier_sem, N_DEV - 1)

        rows = lax.broadcasted_iota(jnp.int32, (m_per, nh), 0)

        def local_partial(h):
            xv = x_ref[:, h * nh:(h + 1) * nh]
            vmax = jnp.max(xv, axis=0, keepdims=True)
            masked = jnp.where(xv == vmax, rows, jnp.int32(N_DEV * m_per))
            lidx = jnp.min(masked, axis=0, keepdims=True)
            gidx = (lidx + my_pos * m_per).astype(jnp.float32)
            comm_ref[0, :, h * nh:(h + 1) * nh] = jnp.concatenate(
                [vmax, gidx], axis=0
            )

        def start_sends(h):
            rdmas = []
            for d in range(1, N_DEV):
                rdma = pltpu.make_async_remote_copy(
                    src_ref=comm_ref.at[0, :, pl.ds(h * nh, nh)],
                    dst_ref=comm_ref.at[N_DEV - d, :, pl.ds(h * nh, nh)],
                    send_sem=send_sems.at[h, d - 1],
                    recv_sem=recv_sems.at[h, N_DEV - 1 - d],
                    device_id=((my_pos + d) % N_DEV,),
                    device_id_type=pl.DeviceIdType.MESH,
                )
                rdma.start()
                rdmas.append(rdma)
            return rdmas

        def combine(h):
            cols = pl.ds(h * nh, nh)
            best_v = comm_ref[0, 0:1, cols]
            best_i = comm_ref[0, 1:2, cols]
            for s in range(1, N_DEV):
                rv = comm_ref[s, 0:1, cols]
                ri = comm_ref[s, 1:2, cols]
                take = (rv > best_v) | ((rv == best_v) & (ri < best_i))
                best_v = jnp.where(take, rv, best_v)
                best_i = jnp.where(take, ri, best_i)
            out_ref[0:1, cols] = best_v
            out_ref[1:2, cols] = best_i

        local_partial(0)
        rdmas0 = start_sends(0)
        local_partial(1)
        rdmas1 = start_sends(1)
        for r in rdmas0:
            r.wait_recv()
        combine(0)
        for r in rdmas1:
            r.wait_recv()
        combine(1)
        for r in rdmas0 + rdmas1:
            r.wait_send()

    return pl.pallas_call(
        body,
        out_shape=jax.ShapeDtypeStruct((2, n), jnp.float32),
        in_specs=[pl.BlockSpec(memory_space=pltpu.VMEM)],
        out_specs=pl.BlockSpec(memory_space=pltpu.VMEM),
        scratch_shapes=[
            pltpu.VMEM((N_DEV, 2, n), jnp.float32),
            pltpu.SemaphoreType.DMA((N_HALF, N_DEV - 1)),
            pltpu.SemaphoreType.DMA((N_HALF, N_DEV - 1)),
        ],
        compiler_params=pltpu.CompilerParams(collective_id=0),
    )(x)
